# baseline (speedup 1.0000x reference)
import functools

import jax
import jax.numpy as jnp
from jax import lax
from jax.experimental import pallas as pl
from jax.experimental.pallas import tpu as pltpu
from jax.experimental.pallas import tpu_sc as plsc

BATCH = 16384
HIDDEN = 64
NUM_WORKERS = 32
ROWS_PER_WORKER = BATCH // NUM_WORKERS
CHUNK = 128
NUM_CHUNKS = ROWS_PER_WORKER // CHUNK


def _gather_kernel(table_hbm, idx_hbm, out_hbm, idx_v, rows_v, sem):
    wid = lax.axis_index("s") * 2 + lax.axis_index("c")
    base = wid * ROWS_PER_WORKER
    pltpu.sync_copy(idx_hbm.at[wid], idx_v)
    copies = []
    for c in range(NUM_CHUNKS):
        copies.append(pltpu.async_copy(
            table_hbm.at[idx_v.at[c]],
            rows_v.at[pl.ds(c * CHUNK, CHUNK)],
            sem))
    for cp in copies:
        cp.wait()
    pltpu.sync_copy(rows_v, out_hbm.at[pl.ds(base, ROWS_PER_WORKER)])


@functools.partial(jax.jit, static_argnums=())
def _embed(labels_3d, table):
    mesh = plsc.VectorSubcoreMesh(core_axis_name="c", subcore_axis_name="s")
    run = functools.partial(
        pl.kernel,
        mesh=mesh,
        out_type=jax.ShapeDtypeStruct((BATCH, HIDDEN), jnp.float32),
        scratch_types=[
            pltpu.VMEM((NUM_CHUNKS, CHUNK), jnp.int32),
            pltpu.VMEM((ROWS_PER_WORKER, HIDDEN), jnp.float32),
            pltpu.SemaphoreType.DMA,
        ],
        compiler_params=pltpu.CompilerParams(use_tc_tiling_on_sc=False),
    )(_gather_kernel)
    return run(table, labels_3d)


def kernel(labels, train, table):
    del train
    labels_3d = labels.astype(jnp.int32).reshape(
        NUM_WORKERS, NUM_CHUNKS, CHUNK)
    return _embed(labels_3d, table)


# speedup vs baseline: 1.1098x; 1.1098x over previous
import functools

import jax
import jax.numpy as jnp
from jax import lax
from jax.experimental import pallas as pl
from jax.experimental.pallas import tpu as pltpu
from jax.experimental.pallas import tpu_sc as plsc

BATCH = 16384
HIDDEN = 64
PADDED = 128
NUM_WORKERS = 32
ROWS_PER_WORKER = BATCH // NUM_WORKERS
CHUNK = 128
NUM_CHUNKS = ROWS_PER_WORKER // CHUNK


def _gather_kernel(table_hbm, idx_hbm, out_hbm, idx_v, rows_v, sem):
    wid = lax.axis_index("s") * 2 + lax.axis_index("c")
    base = wid * ROWS_PER_WORKER
    pltpu.sync_copy(idx_hbm.at[wid], idx_v)
    copies = []
    for c in range(NUM_CHUNKS):
        copies.append(pltpu.async_copy(
            table_hbm.at[idx_v.at[c]],
            rows_v.at[pl.ds(c * CHUNK, CHUNK)],
            sem))
    for cp in copies:
        cp.wait()
    pltpu.sync_copy(rows_v.at[:, pl.ds(0, HIDDEN)],
                    out_hbm.at[pl.ds(base, ROWS_PER_WORKER)])


@jax.jit
def _embed(labels_3d, table_padded):
    mesh = plsc.VectorSubcoreMesh(core_axis_name="c", subcore_axis_name="s")
    run = functools.partial(
        pl.kernel,
        mesh=mesh,
        out_type=jax.ShapeDtypeStruct((BATCH, HIDDEN), jnp.float32),
        scratch_types=[
            pltpu.VMEM((NUM_CHUNKS, CHUNK), jnp.int32),
            pltpu.VMEM((ROWS_PER_WORKER, PADDED), jnp.float32),
            pltpu.SemaphoreType.DMA,
        ],
        compiler_params=pltpu.CompilerParams(use_tc_tiling_on_sc=False),
    )(_gather_kernel)
    return run(table_padded, labels_3d)


def kernel(labels, train, table):
    del train
    labels_3d = labels.astype(jnp.int32).reshape(
        NUM_WORKERS, NUM_CHUNKS, CHUNK)
    table_padded = jnp.pad(table, ((0, 0), (0, PADDED - HIDDEN)))
    return _embed(labels_3d, table_padded)


# speedup vs baseline: 3.4792x; 3.1350x over previous
import functools

import jax
import jax.numpy as jnp
from jax import lax
from jax.experimental import pallas as pl
from jax.experimental.pallas import tpu as pltpu
from jax.experimental.pallas import tpu_sc as plsc

BATCH = 16384
HIDDEN = 64
LANES = 128
NUM_WORKERS = 32
RPW = BATCH // NUM_WORKERS
NCH = RPW // LANES


def _gather_kernel(tabT, lab_hbm, pos_hbm, out_hbm,
                   labv, posv, rectring, stage, rsv, rsem, osem):
    wid = lax.axis_index("s") * 2 + lax.axis_index("c")
    pltpu.sync_copy(lab_hbm.at[wid], labv)
    pltpu.sync_copy(pos_hbm.at[wid], posv)

    iot = lax.iota(jnp.int32, 16)

    def lab_at(i):
        grp = (i >> 4) & 7
        v = labv[i >> 7, pl.ds(pl.multiple_of(grp * 16, 16), 16)]
        return jnp.sum(jnp.where(iot == (i & 15), v, 0))

    def rd(i, carry):
        cnt, prev = carry
        tc = lab_at(i) >> 7
        new = tc != prev

        @pl.when(new)
        def _():
            rsv[cnt] = i

        return cnt + new.astype(jnp.int32), tc

    nruns, _ = lax.fori_loop(0, RPW, rd, (jnp.int32(0), jnp.int32(-1)))
    rsv[nruns] = jnp.int32(RPW)

    def start_rect(i_first, b):
        start = pl.multiple_of((lab_at(i_first) >> 7) * LANES, LANES)
        return pltpu.async_copy(
            tabT.at[:, pl.ds(start, LANES)], rectring.at[b], rsem.at[b])

    start_rect(jnp.int32(0), jnp.int32(0))

    def run_body(r, _):
        p = r & 1

        @pl.when(r + 1 < nruns)
        def _():
            start_rect(rsv[r + 1], 1 - p)

        pltpu.make_async_copy(
            tabT.at[:, pl.ds(0, LANES)], rectring.at[p], rsem.at[p]).wait()

        def ext(i, _):
            lane = jnp.broadcast_to(lab_at(i) & (LANES - 1), (16,))
            for j in range(HIDDEN // 16):
                g = plsc.load_gather(rectring.at[p], [iot + j * 16, lane])
                stage[i, pl.ds(j * 16, 16)] = g
            return 0

        lax.fori_loop(rsv[r], rsv[r + 1], ext, 0)
        return 0

    lax.fori_loop(0, nruns, run_body, 0)

    cps = []
    for c in range(NCH):
        for k in range(LANES // 16):
            pv = posv[c, pl.ds(k * 16, 16)]
            cps.append(pltpu.async_copy(
                stage.at[pl.ds(c * LANES + k * 16, 16)], out_hbm.at[pv], osem))
    for cp in cps:
        cp.wait()


@jax.jit
def _embed(lab3, pos3, tabT):
    mesh = plsc.VectorSubcoreMesh(core_axis_name="c", subcore_axis_name="s")
    run = functools.partial(
        pl.kernel,
        mesh=mesh,
        out_type=jax.ShapeDtypeStruct((BATCH, LANES), jnp.float32),
        scratch_types=[
            pltpu.VMEM((NCH, LANES), jnp.int32),
            pltpu.VMEM((NCH, LANES), jnp.int32),
            pltpu.VMEM((2, HIDDEN, LANES), jnp.float32),
            pltpu.VMEM((RPW, LANES), jnp.float32),
            pltpu.SMEM((RPW + 32, ), jnp.int32),
            pltpu.SemaphoreType.DMA((2,)),
            pltpu.SemaphoreType.DMA,
        ],
        compiler_params=pltpu.CompilerParams(
            disable_bounds_checks=True, needs_layout_passes=False),
    )(_gather_kernel)
    return run(tabT, lab3, pos3)


def kernel(labels, train, table):
    del train
    order = lax.iota(jnp.int32, BATCH)
    slab, order = lax.sort((labels.astype(jnp.int32), order), num_keys=1)
    out_p = _embed(slab.reshape(NUM_WORKERS, NCH, LANES),
                   order.reshape(NUM_WORKERS, NCH, LANES),
                   table.T)
    return out_p[:, :HIDDEN]


# speedup vs baseline: 4.8396x; 1.3910x over previous
import functools

import jax
import jax.numpy as jnp
from jax import lax
from jax.experimental import pallas as pl
from jax.experimental.pallas import tpu as pltpu
from jax.experimental.pallas import tpu_sc as plsc

BATCH = 16384
HIDDEN = 64
LANES = 128
NUM_WORKERS = 32
RPW = BATCH // NUM_WORKERS
NCH = RPW // LANES
NBUF = 4


def _gather_kernel(tabT, lab_hbm, pos_hbm, out_hbm,
                   labv, posv, rectring, stage, rsv, rsem, osem):
    wid = lax.axis_index("s") * 2 + lax.axis_index("c")
    pltpu.sync_copy(lab_hbm.at[wid], labv)
    pltpu.sync_copy(pos_hbm.at[wid], posv)

    iot = lax.iota(jnp.int32, 16)

    def lab_splat(i):
        grp = (i >> 4) & 7
        v = labv[i >> 7, pl.ds(pl.multiple_of(grp * 16, 16), 16)]
        idx = jnp.broadcast_to(i & 15, (16,))
        return lax.gather(
            v, idx[:, None],
            dimension_numbers=lax.GatherDimensionNumbers(
                offset_dims=(), collapsed_slice_dims=(0,),
                start_index_map=(0,)),
            slice_sizes=(1,),
            mode=lax.GatherScatterMode.PROMISE_IN_BOUNDS)

    def lab_at(i):
        return jnp.sum(jnp.where(iot == 0, lab_splat(i), 0))

    def start_rect(i_first, b):
        start = pl.multiple_of((lab_at(i_first) >> 7) * LANES, LANES)
        return pltpu.async_copy(
            tabT.at[:, pl.ds(start, LANES)], rectring.at[b], rsem.at[b])

    start_rect(jnp.int32(0), jnp.int32(0))

    def rd(i, carry):
        cnt, prev = carry
        tc = lab_at(i) >> 7
        new = tc != prev

        @pl.when(new)
        def _():
            rsv[cnt] = i

        return cnt + new.astype(jnp.int32), tc

    nruns, _ = lax.fori_loop(0, RPW, rd, (jnp.int32(0), jnp.int32(-1)))
    rsv[nruns] = jnp.int32(RPW)

    for b in range(1, NBUF):
        @pl.when(b < nruns)
        def _(b=b):
            start_rect(rsv[b], jnp.int32(b))

    def run_body(r, _):
        p = r & (NBUF - 1)
        pltpu.make_async_copy(
            tabT.at[:, pl.ds(0, LANES)], rectring.at[p], rsem.at[p]).wait()

        def ext(i, _):
            lane = lab_splat(i) & (LANES - 1)
            for j in range(HIDDEN // 16):
                g = plsc.load_gather(rectring.at[p], [iot + j * 16, lane])
                stage[i, pl.ds(j * 16, 16)] = g
            return 0

        lax.fori_loop(rsv[r], rsv[r + 1], ext, 0)

        @pl.when(r + NBUF < nruns)
        def _():
            start_rect(rsv[r + NBUF], p)

        return 0

    lax.fori_loop(0, nruns, run_body, 0)

    cps = []
    for c in range(NCH):
        for k in range(LANES // 16):
            pv = posv[c, pl.ds(k * 16, 16)]
            cps.append(pltpu.async_copy(
                stage.at[pl.ds(c * LANES + k * 16, 16)], out_hbm.at[pv], osem))
    for cp in cps:
        cp.wait()


@jax.jit
def _embed(lab3, pos3, tabT):
    mesh = plsc.VectorSubcoreMesh(core_axis_name="c", subcore_axis_name="s")
    run = functools.partial(
        pl.kernel,
        mesh=mesh,
        out_type=jax.ShapeDtypeStruct((BATCH, LANES), jnp.float32),
        scratch_types=[
            pltpu.VMEM((NCH, LANES), jnp.int32),
            pltpu.VMEM((NCH, LANES), jnp.int32),
            pltpu.VMEM((NBUF, HIDDEN, LANES), jnp.float32),
            pltpu.VMEM((RPW, LANES), jnp.float32),
            pltpu.SMEM((RPW + 32, ), jnp.int32),
            pltpu.SemaphoreType.DMA((NBUF,)),
            pltpu.SemaphoreType.DMA,
        ],
        compiler_params=pltpu.CompilerParams(
            disable_bounds_checks=True, needs_layout_passes=False),
    )(_gather_kernel)
    return run(tabT, lab3, pos3)


def kernel(labels, train, table):
    del train
    order = lax.iota(jnp.int32, BATCH)
    slab, order = lax.sort((labels.astype(jnp.int32), order), num_keys=1)
    out_p = _embed(slab.reshape(NUM_WORKERS, NCH, LANES),
                   order.reshape(NUM_WORKERS, NCH, LANES),
                   table.T)
    return out_p[:, :HIDDEN]


# speedup vs baseline: 4.9900x; 1.0311x over previous
import functools

import jax
import jax.numpy as jnp
from jax import lax
from jax.experimental import pallas as pl
from jax.experimental.pallas import tpu as pltpu
from jax.experimental.pallas import tpu_sc as plsc

BATCH = 16384
HIDDEN = 64
LANES = 128
NUM_WORKERS = 32
RPW = BATCH // NUM_WORKERS
NCH = RPW // LANES
NBUF = 6


def _gather_kernel(tabT, lab_hbm, pos_hbm, out_hbm,
                   labv, posv, rectring, stage, rsv, rsem, osem):
    wid = lax.axis_index("s") * 2 + lax.axis_index("c")
    pltpu.sync_copy(lab_hbm.at[wid], labv)
    pltpu.sync_copy(pos_hbm.at[wid], posv)

    iot = lax.iota(jnp.int32, 16)

    def lab_splat(i):
        grp = (i >> 4) & 7
        v = labv[i >> 7, pl.ds(pl.multiple_of(grp * 16, 16), 16)]
        idx = jnp.broadcast_to(i & 15, (16,))
        return lax.gather(
            v, idx[:, None],
            dimension_numbers=lax.GatherDimensionNumbers(
                offset_dims=(), collapsed_slice_dims=(0,),
                start_index_map=(0,)),
            slice_sizes=(1,),
            mode=lax.GatherScatterMode.PROMISE_IN_BOUNDS)

    def lab_at(i):
        return jnp.sum(jnp.where(iot == 0, lab_splat(i), 0))

    def start_rect_tc(tc, b):
        start = pl.multiple_of(tc * LANES, LANES)
        return pltpu.async_copy(
            tabT.at[:, pl.ds(start, LANES)], rectring.at[b], rsem.at[b])

    def start_rect(i_first, b):
        return start_rect_tc(lab_at(i_first) >> 7, b)

    def rd(i, carry):
        cnt, prev = carry
        tc = lab_at(i) >> 7
        new = tc != prev

        @pl.when(new)
        def _():
            rsv[cnt] = i

            @pl.when(cnt < NBUF)
            def _():
                start_rect_tc(tc, cnt)

        return cnt + new.astype(jnp.int32), tc

    nruns, _ = lax.fori_loop(0, RPW, rd, (jnp.int32(0), jnp.int32(-1)))
    rsv[nruns] = jnp.int32(RPW)

    def run_body(r, _):
        p = lax.rem(r, jnp.int32(NBUF))
        pltpu.make_async_copy(
            tabT.at[:, pl.ds(0, LANES)], rectring.at[p], rsem.at[p]).wait()

        def ext(i, _):
            lane = lab_splat(i) & (LANES - 1)
            for j in range(HIDDEN // 16):
                g = plsc.load_gather(rectring.at[p], [iot + j * 16, lane])
                stage[i, pl.ds(j * 16, 16)] = g
            return 0

        lax.fori_loop(rsv[r], rsv[r + 1], ext, 0)

        @pl.when(r + NBUF < nruns)
        def _():
            start_rect(rsv[r + NBUF], p)

        return 0

    lax.fori_loop(0, nruns, run_body, 0)

    cps = []
    for c in range(NCH):
        for k in range(LANES // 16):
            pv = posv[c, pl.ds(k * 16, 16)]
            cps.append(pltpu.async_copy(
                stage.at[pl.ds(c * LANES + k * 16, 16)], out_hbm.at[pv], osem))
    for cp in cps:
        cp.wait()


@jax.jit
def _embed(lab3, pos3, tabT):
    mesh = plsc.VectorSubcoreMesh(core_axis_name="c", subcore_axis_name="s")
    run = functools.partial(
        pl.kernel,
        mesh=mesh,
        out_type=jax.ShapeDtypeStruct((BATCH, LANES), jnp.float32),
        scratch_types=[
            pltpu.VMEM((NCH, LANES), jnp.int32),
            pltpu.VMEM((NCH, LANES), jnp.int32),
            pltpu.VMEM((NBUF, HIDDEN, LANES), jnp.float32),
            pltpu.VMEM((RPW, LANES), jnp.float32),
            pltpu.SMEM((RPW + 32, ), jnp.int32),
            pltpu.SemaphoreType.DMA((NBUF,)),
            pltpu.SemaphoreType.DMA,
        ],
        compiler_params=pltpu.CompilerParams(
            disable_bounds_checks=True, needs_layout_passes=False),
    )(_gather_kernel)
    return run(tabT, lab3, pos3)


def kernel(labels, train, table):
    del train
    order = lax.iota(jnp.int32, BATCH)
    slab, order = lax.sort((labels.astype(jnp.int32), order), num_keys=1)
    out_p = _embed(slab.reshape(NUM_WORKERS, NCH, LANES),
                   order.reshape(NUM_WORKERS, NCH, LANES),
                   table.T)
    return out_p[:, :HIDDEN]
